# baseline (speedup 1.0000x reference)
import functools

import jax
import jax.numpy as jnp
from jax import lax
from jax.experimental import pallas as pl
from jax.experimental.pallas import tpu as pltpu
from jax.experimental.pallas import tpu_sc as plsc

D_MODEL = 64
SCALE = 8.0
_S = 4


@functools.lru_cache(maxsize=None)
def _make_tc_prep(V: int):
    BW = 128

    def body(a_ref, o_ref):
        t = a_ref[...].T.reshape(BW // 2, 2, D_MODEL) * SCALE
        o_ref[:, :D_MODEL] = t[:, 0, :]
        o_ref[:, D_MODEL:] = t[:, 1, :]

    grid = (V + BW - 1) // BW
    return pl.pallas_call(
        body,
        grid=(grid,),
        in_specs=[pl.BlockSpec((D_MODEL, BW), lambda i: (0, i))],
        out_specs=pl.BlockSpec((BW // 2, 2 * D_MODEL), lambda i: (i, 0)),
        out_shape=jax.ShapeDtypeStruct((V // 2, 2 * D_MODEL), jnp.float32),
    )


@functools.lru_cache(maxsize=None)
def _make_sc_gather(R: int, C: int, V: int):
    info = plsc.get_sparse_core_info()
    NC, NS = info.num_cores, info.num_subcores
    NW = NC * NS
    assert R % (NW * _S) == 0, R
    chunks_per_w = R // (NW * _S)
    xrows_per_w = R // NW
    splits = []
    o = 0
    while o < C:
        splits.append((o, min(128, C - o)))
        o += 128

    mesh = plsc.VectorSubcoreMesh(core_axis_name="c", subcore_axis_name="s")

    @functools.partial(
        pl.kernel,
        mesh=mesh,
        out_type=jax.ShapeDtypeStruct((R, C, D_MODEL), jnp.float32),
        scratch_types=[
            pltpu.VMEM((2, _S, C), jnp.int32),
            pltpu.VMEM((2, _S, C, D_MODEL), jnp.float32),
            pltpu.SemaphoreType.DMA,
            pltpu.SemaphoreType.DMA,
        ],
        compiler_params=pltpu.CompilerParams(use_tc_tiling_on_sc=False),
    )
    def k(emb_hbm, idx_hbm, out_hbm, idx_v, rows_v, sem0, sem1):
        sems = (sem0, sem1)
        wid = lax.axis_index("s") * NC + lax.axis_index("c")
        w_xrow0 = wid * xrows_per_w

        def fire(c, p):
            xrow = w_xrow0 + c * _S
            pltpu.sync_copy(idx_hbm.at[pl.ds(xrow, _S)], idx_v.at[p])
            for r in range(_S):
                for o, n in splits:
                    pltpu.async_copy(
                        emb_hbm.at[idx_v.at[p, r, pl.ds(o, n)]],
                        rows_v.at[p, r, pl.ds(o, n)],
                        sems[p],
                    )

        def drain(p):
            for r in range(_S):
                for o, n in splits:
                    pltpu.make_async_copy(
                        emb_hbm.at[idx_v.at[p, r, pl.ds(o, n)]],
                        rows_v.at[p, r, pl.ds(o, n)],
                        sems[p],
                    ).wait()

        def finish(c, p):
            drain(p)
            xrow = w_xrow0 + c * _S
            pltpu.sync_copy(rows_v.at[p], out_hbm.at[pl.ds(xrow, _S)])

        fire(0, 0)

        def pair_body(i, carry):
            for b in range(2):
                c = 2 * i + b
                fire(c + 1, 1 - b)
                finish(c, b)
            return carry

        n = chunks_per_w
        lax.fori_loop(0, (n - 1) // 2, pair_body, 0)
        if n % 2 == 1:
            finish(n - 1, (n - 1) % 2)
        else:
            fire(n - 1, (n - 1) % 2)
            finish(n - 2, (n - 2) % 2)
            finish(n - 1, (n - 1) % 2)

    return k


def kernel(x, emb):
    xi = x.astype(jnp.int32)
    V = emb.shape[0]
    embL = _make_tc_prep(V)(emb.T).reshape(V, D_MODEL)
    return _make_sc_gather(x.shape[0], x.shape[1], V)(embL, xi)


# speedup vs baseline: 2.5763x; 2.5763x over previous
import functools

import jax
import jax.numpy as jnp
from jax import lax
from jax.experimental import pallas as pl
from jax.experimental.pallas import tpu as pltpu
from jax.experimental.pallas import tpu_sc as plsc

D_MODEL = 64
SCALE = 8.0
_BW = 128


@functools.lru_cache(maxsize=None)
def _make_sc_gather(R: int, C: int, V: int):
    info = plsc.get_sparse_core_info()
    NC, NS = info.num_cores, info.num_subcores
    NW = NC * NS
    assert R == NW * _BW, (R, NW)
    BH = R // _BW
    CP = (C + 15) // 16 * 16

    mesh = plsc.VectorSubcoreMesh(core_axis_name="c", subcore_axis_name="s")

    @functools.partial(
        pl.kernel,
        mesh=mesh,
        out_type=jax.ShapeDtypeStruct((C, 8, BH, 8, _BW), jnp.float32),
        scratch_types=[
            pltpu.VMEM((_BW, CP), jnp.int32),
            pltpu.VMEM((C, _BW), jnp.int32),
            pltpu.VMEM((2, _BW, D_MODEL), jnp.float32),
            pltpu.VMEM((2, 8, 8, _BW), jnp.float32),
            pltpu.SemaphoreType.DMA,
            pltpu.SemaphoreType.DMA,
        ],
        compiler_params=pltpu.CompilerParams(
            use_tc_tiling_on_sc=False, needs_layout_passes=False),
    )
    def k(emb_hbm, idx_hbm, out_hbm, x_v, xT_v, g_v, o_v, sem0, sem1):
        sems = (sem0, sem1)
        wid = lax.axis_index("s") * NC + lax.axis_index("c")
        row0 = wid * _BW

        pltpu.sync_copy(idx_hbm.at[pl.ds(row0, _BW)], x_v.at[:, pl.ds(0, C)])

        iota = lax.iota(jnp.int32, 16)

        def xpose_body(r, carry):
            rr = jnp.broadcast_to(r, (16,))
            for j in range(CP // 16):
                vals = x_v[r, pl.ds(j * 16, 16)]
                cols = iota + (j * 16)
                if (j + 1) * 16 <= C:
                    plsc.store_scatter(xT_v, [cols, rr], vals)
                else:
                    plsc.store_scatter(xT_v, [cols, rr], vals,
                                       mask=cols < C)
            return carry

        lax.fori_loop(0, _BW, xpose_body, 0)

        def fire(c, p):
            pltpu.async_copy(emb_hbm.at[xT_v.at[c]], g_v.at[p], sems[p])

        def drain(c, p):
            pltpu.make_async_copy(
                emb_hbm.at[xT_v.at[c]], g_v.at[p], sems[p]).wait()

        def finish(c, p):
            drain(c, p)

            def tr_body(bl, carry):
                bb = jnp.broadcast_to(bl, (16,))
                for j in range(D_MODEL // 16):
                    vals = g_v[p, bl, pl.ds(j * 16, 16)] * SCALE
                    d = iota + (j * 16)
                    plsc.store_scatter(
                        o_v.at[p], [lax.shift_right_logical(d, 3),
                                    lax.bitwise_and(d, 7), bb], vals)
                return carry

            lax.fori_loop(0, _BW, tr_body, 0)

            for dh in range(8):
                pltpu.sync_copy(o_v.at[p, dh], out_hbm.at[c, dh, wid])

        fire(0, 0)

        def pair_body(i, carry):
            for b in range(2):
                c = 2 * i + b
                fire(c + 1, 1 - b)
                finish(c, b)
            return carry

        lax.fori_loop(0, (C - 1) // 2, pair_body, 0)
        if C % 2 == 1:
            finish(C - 1, (C - 1) % 2)
        else:
            fire(C - 1, (C - 1) % 2)
            finish(C - 2, (C - 2) % 2)
            finish(C - 1, (C - 1) % 2)

    return k


def kernel(x, emb):
    xi = x.astype(jnp.int32)
    R, C = x.shape
    out5 = _make_sc_gather(R, C, emb.shape[0])(emb, xi)
    return out5.transpose(2, 4, 0, 1, 3).reshape(R, C, D_MODEL)


# speedup vs baseline: 4.0145x; 1.5583x over previous
import functools

import jax
import jax.numpy as jnp
from jax import lax
from jax.experimental import pallas as pl
from jax.experimental.pallas import tpu as pltpu
from jax.experimental.pallas import tpu_sc as plsc

D_MODEL = 64
SCALE = 8.0
_S = 4


@functools.lru_cache(maxsize=None)
def _make_tc_prep(V: int):
    BW = 2048

    def body(a_ref, o_ref):
        t = a_ref[...].T.reshape(BW // 2, 2, D_MODEL) * SCALE
        o_ref[:, :D_MODEL] = t[:, 0, :]
        o_ref[:, D_MODEL:] = t[:, 1, :]

    grid = (V + BW - 1) // BW
    return pl.pallas_call(
        body,
        grid=(grid,),
        in_specs=[pl.BlockSpec((D_MODEL, BW), lambda i: (0, i))],
        out_specs=pl.BlockSpec((BW // 2, 2 * D_MODEL), lambda i: (i, 0)),
        out_shape=jax.ShapeDtypeStruct((V // 2, 2 * D_MODEL), jnp.float32),
    )


@functools.lru_cache(maxsize=None)
def _make_sc_gather(R: int, C: int, V: int):
    info = plsc.get_sparse_core_info()
    NC, NS = info.num_cores, info.num_subcores
    NW = NC * NS
    assert R % (NW * _S) == 0, R
    chunks_per_w = R // (NW * _S)
    xrows_per_w = R // NW
    splits = []
    o = 0
    while o < C:
        splits.append((o, min(128, C - o)))
        o += 128

    mesh = plsc.VectorSubcoreMesh(core_axis_name="c", subcore_axis_name="s")

    @functools.partial(
        pl.kernel,
        mesh=mesh,
        out_type=jax.ShapeDtypeStruct((R, C, D_MODEL), jnp.float32),
        scratch_types=[
            pltpu.VMEM((2, _S, C), jnp.int32),
            pltpu.VMEM((2, _S, C, D_MODEL), jnp.float32),
            pltpu.SemaphoreType.DMA,
            pltpu.SemaphoreType.DMA,
        ],
        compiler_params=pltpu.CompilerParams(use_tc_tiling_on_sc=False),
    )
    def k(emb_hbm, idx_hbm, out_hbm, idx_v, rows_v, sem0, sem1):
        sems = (sem0, sem1)
        wid = lax.axis_index("s") * NC + lax.axis_index("c")
        w_xrow0 = wid * xrows_per_w

        def fire(c, p):
            xrow = w_xrow0 + c * _S
            pltpu.sync_copy(idx_hbm.at[pl.ds(xrow, _S)], idx_v.at[p])
            for r in range(_S):
                for o, n in splits:
                    pltpu.async_copy(
                        emb_hbm.at[idx_v.at[p, r, pl.ds(o, n)]],
                        rows_v.at[p, r, pl.ds(o, n)],
                        sems[p],
                    )

        def drain(p):
            for r in range(_S):
                for o, n in splits:
                    pltpu.make_async_copy(
                        emb_hbm.at[idx_v.at[p, r, pl.ds(o, n)]],
                        rows_v.at[p, r, pl.ds(o, n)],
                        sems[p],
                    ).wait()

        def finish(c, p):
            drain(p)
            xrow = w_xrow0 + c * _S
            pltpu.sync_copy(rows_v.at[p], out_hbm.at[pl.ds(xrow, _S)])

        fire(0, 0)

        def pair_body(i, carry):
            for b in range(2):
                c = 2 * i + b
                fire(c + 1, 1 - b)
                finish(c, b)
            return carry

        n = chunks_per_w
        lax.fori_loop(0, (n - 1) // 2, pair_body, 0)
        if n % 2 == 1:
            finish(n - 1, (n - 1) % 2)
        else:
            fire(n - 1, (n - 1) % 2)
            finish(n - 2, (n - 2) % 2)
            finish(n - 1, (n - 1) % 2)

    return k


def kernel(x, emb):
    xi = x.astype(jnp.int32)
    V = emb.shape[0]
    embL = _make_tc_prep(V)(emb.T).reshape(V, D_MODEL)
    return _make_sc_gather(x.shape[0], x.shape[1], V)(embL, xi)


# speedup vs baseline: 4.3937x; 1.0945x over previous
import functools

import jax
import jax.numpy as jnp
from jax import lax
from jax.experimental import pallas as pl
from jax.experimental.pallas import tpu as pltpu
from jax.experimental.pallas import tpu_sc as plsc

D_MODEL = 64
SCALE = 8.0
_S = 4


@functools.lru_cache(maxsize=None)
def _make_tc_prep(V: int):
    BW = 8192

    def body(a_ref, o_ref):
        t = a_ref[...].T.reshape(BW // 2, 2, D_MODEL) * SCALE
        o_ref[:, :D_MODEL] = t[:, 0, :]
        o_ref[:, D_MODEL:] = t[:, 1, :]

    grid = (V + BW - 1) // BW
    return pl.pallas_call(
        body,
        grid=(grid,),
        in_specs=[pl.BlockSpec((D_MODEL, BW), lambda i: (0, i))],
        out_specs=pl.BlockSpec((BW // 2, 2 * D_MODEL), lambda i: (i, 0)),
        out_shape=jax.ShapeDtypeStruct((V // 2, 2 * D_MODEL), jnp.float32),
    )


@functools.lru_cache(maxsize=None)
def _make_sc_gather(R: int, C: int, V: int):
    info = plsc.get_sparse_core_info()
    NC, NS = info.num_cores, info.num_subcores
    NW = NC * NS
    assert R % (NW * _S) == 0, R
    chunks_per_w = R // (NW * _S)
    xrows_per_w = R // NW
    splits = []
    o = 0
    while o < C:
        splits.append((o, min(128, C - o)))
        o += 128

    mesh = plsc.VectorSubcoreMesh(core_axis_name="c", subcore_axis_name="s")

    @functools.partial(
        pl.kernel,
        mesh=mesh,
        out_type=jax.ShapeDtypeStruct((R, C, D_MODEL), jnp.float32),
        scratch_types=[
            pltpu.VMEM((2, _S, C), jnp.int32),
            pltpu.VMEM((2, _S, C, D_MODEL), jnp.float32),
            pltpu.SemaphoreType.DMA,
            pltpu.SemaphoreType.DMA,
        ],
        compiler_params=pltpu.CompilerParams(use_tc_tiling_on_sc=False),
    )
    def k(emb_hbm, idx_hbm, out_hbm, idx_v, rows_v, sem0, sem1):
        sems = (sem0, sem1)
        wid = lax.axis_index("s") * NC + lax.axis_index("c")
        w_xrow0 = wid * xrows_per_w

        def fire(c, p):
            xrow = w_xrow0 + c * _S
            pltpu.sync_copy(idx_hbm.at[pl.ds(xrow, _S)], idx_v.at[p])
            for r in range(_S):
                for o, n in splits:
                    pltpu.async_copy(
                        emb_hbm.at[idx_v.at[p, r, pl.ds(o, n)]],
                        rows_v.at[p, r, pl.ds(o, n)],
                        sems[p],
                    )

        def drain(p):
            for r in range(_S):
                for o, n in splits:
                    pltpu.make_async_copy(
                        emb_hbm.at[idx_v.at[p, r, pl.ds(o, n)]],
                        rows_v.at[p, r, pl.ds(o, n)],
                        sems[p],
                    ).wait()

        def finish(c, p):
            drain(p)
            xrow = w_xrow0 + c * _S
            pltpu.sync_copy(rows_v.at[p], out_hbm.at[pl.ds(xrow, _S)])

        fire(0, 0)

        def pair_body(i, carry):
            for b in range(2):
                c = 2 * i + b
                fire(c + 1, 1 - b)
                finish(c, b)
            return carry

        n = chunks_per_w
        lax.fori_loop(0, (n - 1) // 2, pair_body, 0)
        if n % 2 == 1:
            finish(n - 1, (n - 1) % 2)
        else:
            fire(n - 1, (n - 1) % 2)
            finish(n - 2, (n - 2) % 2)
            finish(n - 1, (n - 1) % 2)

    return k


def kernel(x, emb):
    xi = x.astype(jnp.int32)
    V = emb.shape[0]
    embL = _make_tc_prep(V)(emb.T).reshape(V, D_MODEL)
    return _make_sc_gather(x.shape[0], x.shape[1], V)(embL, xi)
